# baseline (speedup 1.0000x reference)
import functools

import jax
import jax.numpy as jnp
from jax import lax
from jax.experimental import pallas as pl
from jax.experimental.pallas import tpu as pltpu
from jax.experimental.pallas import tpu_sc as plsc

N_NODES = 10000
FEATS = 128
N_EDGES = 320000
EPS = 1e-5

NC = 2
NS = 16
NW = NC * NS
CHUNK = 128
SLOW_C = 0
CH_A = 56
CH_B = 104
CMAX = max(CH_A, CH_B)
NPADF = 10016
ROWS = 10112
RPT = ROWS // NS
IBUF = 4


def _leaky_relu_tc(x):
    def body(x_ref, o_ref):
        v = x_ref[...]
        o_ref[...] = jnp.where(v > 0, v, jnp.float32(0.2) * v)

    return pl.pallas_call(
        body,
        out_shape=jax.ShapeDtypeStruct(x.shape, x.dtype),
    )(x)


def _sc_segment_sum(h, ei4, zrows):
    mesh = plsc.VectorSubcoreMesh(core_axis_name="c", subcore_axis_name="s")

    @functools.partial(
        pl.kernel,
        mesh=mesh,
        out_type=jax.ShapeDtypeStruct((NC, ROWS, FEATS), jnp.float32),
        scratch_types=[pltpu.VMEM((2, CHUNK), jnp.int32) for _ in range(IBUF)]
        + [
            pltpu.VMEM((CHUNK, FEATS), jnp.float32),
            pltpu.SemaphoreType.DMA,
        ]
        + [pltpu.SemaphoreType.DMA for _ in range(IBUF)]
        + [pltpu.VMEM_SHARED((ROWS, FEATS), jnp.float32)],
    )
    def k(h_hbm, ei_hbm, z_hbm, out_hbm, i0, i1, i2, i3, rows_v, gsem,
          s0, s1, s2, s3, acc):
        idx = (i0, i1, i2, i3)
        isem = (s0, s1, s2, s3)
        c = lax.axis_index("c")
        s = lax.axis_index("s")
        pltpu.sync_copy(z_hbm, acc.at[pl.ds(s * RPT, RPT)])
        plsc.subcore_barrier()

        def run(base, n_ch):
            for q in range(IBUF):
                pltpu.async_copy(ei_hbm.at[s, base + q], idx[q], isem[q])

            def step(j, u, reload):
                pltpu.make_async_copy(
                    ei_hbm.at[s, base + j], idx[u], isem[u]).wait()
                pltpu.async_copy(h_hbm.at[idx[u].at[0]], rows_v, gsem).wait()
                pltpu.sync_copy(rows_v, acc.at[idx[u].at[1]], add=True)
                if reload:
                    pltpu.async_copy(
                        ei_hbm.at[s, base + j + IBUF], idx[u], isem[u])

            def body(it, carry):
                for u in range(IBUF):
                    step(it * IBUF + u, u, True)
                return carry

            lax.fori_loop(0, n_ch // IBUF - 1, body, 0)
            for u in range(IBUF):
                step(n_ch - IBUF + u, u, False)

        @pl.when(c == SLOW_C)
        def _():
            run(0, CH_A)

        @pl.when(c != SLOW_C)
        def _():
            run(CH_A, CH_B)
        plsc.subcore_barrier()
        pltpu.sync_copy(acc.at[pl.ds(s * RPT, RPT)],
                        out_hbm.at[c, pl.ds(s * RPT, RPT)])

    return k(h, ei4, zrows)


def _tc_finish(p0, p1, wt, b2, g2, be2):
    def body(p0_ref, p1_ref, wt_ref, b_ref, g_ref, be_ref, o_ref):
        agg = p0_ref[...] + p1_ref[...]
        h2 = jnp.dot(agg, wt_ref[...], preferred_element_type=jnp.float32)
        h2 = h2 + b_ref[...]
        mean = jnp.mean(h2, axis=0, keepdims=True)
        ctr = h2 - mean
        var = jnp.mean(ctr * ctr, axis=0, keepdims=True)
        o_ref[...] = g_ref[...] * ctr * lax.rsqrt(var + EPS) + be_ref[...]

    return pl.pallas_call(
        body,
        out_shape=jax.ShapeDtypeStruct((N_NODES, FEATS), jnp.float32),
    )(p0, p1, wt, b2, g2, be2)


def kernel(feature, edge_index, W, b, gamma, beta):
    fx = jnp.pad(feature, ((0, NPADF - N_NODES), (0, 0)))
    h = _leaky_relu_tc(fx)
    ei = edge_index.astype(jnp.int32)
    pad = NS * (CH_A + CH_B) * CHUNK - N_EDGES
    src_p = jnp.concatenate(
        [ei[0], jnp.full((pad,), N_NODES, jnp.int32)]).reshape(
            NS, CH_A + CH_B, 1, CHUNK)
    dummy = jnp.arange(pad, dtype=jnp.int32) % N_NODES
    dst_p = jnp.concatenate([ei[1], dummy]).reshape(NS, CH_A + CH_B, 1, CHUNK)
    ei4 = jnp.concatenate([src_p, dst_p], axis=2)
    zrows = jnp.zeros((RPT, FEATS), jnp.float32)
    parts = _sc_segment_sum(h, ei4, zrows)
    p0 = parts[0, :N_NODES]
    p1 = parts[1, :N_NODES]
    return _tc_finish(p0, p1, W.T,
                      b.reshape(1, FEATS),
                      gamma.reshape(1, FEATS),
                      beta.reshape(1, FEATS))


# speedup vs baseline: 1.0623x; 1.0623x over previous
import functools

import jax
import jax.numpy as jnp
from jax import lax
from jax.experimental import pallas as pl
from jax.experimental.pallas import tpu as pltpu
from jax.experimental.pallas import tpu_sc as plsc

N_NODES = 10000
FEATS = 128
N_EDGES = 320000
EPS = 1e-5

NC = 2
NS = 16
NW = NC * NS
CHUNK = 128
SLOW_C = 1
CH_A = 60
CH_B = 100
NPADF = 10016
ROWS = 10112
RPT = ROWS // NS


def _leaky_relu_tc(x):
    def body(x_ref, o_ref):
        v = x_ref[...]
        o_ref[...] = jnp.where(v > 0, v, jnp.float32(0.2) * v)

    return pl.pallas_call(
        body,
        out_shape=jax.ShapeDtypeStruct(x.shape, x.dtype),
    )(x)


def _sc_segment_sum(h, ei4, zrows):
    mesh = plsc.VectorSubcoreMesh(core_axis_name="c", subcore_axis_name="s")

    @functools.partial(
        pl.kernel,
        mesh=mesh,
        out_type=jax.ShapeDtypeStruct((NC, ROWS, FEATS), jnp.float32),
        scratch_types=[
            pltpu.VMEM((CH_B, 2, CHUNK), jnp.int32),
            pltpu.VMEM((CHUNK, FEATS), jnp.float32),
            pltpu.SemaphoreType.DMA,
            pltpu.VMEM_SHARED((ROWS, FEATS), jnp.float32),
        ],
    )
    def k(h_hbm, ei_hbm, z_hbm, out_hbm, idx_v, rows_v, gsem, acc):
        c = lax.axis_index("c")
        s = lax.axis_index("s")
        pltpu.sync_copy(z_hbm, acc.at[pl.ds(s * RPT, RPT)])

        def run(base, n_ch):
            pltpu.sync_copy(ei_hbm.at[s, pl.ds(base, n_ch)],
                            idx_v.at[pl.ds(0, n_ch)])
            plsc.subcore_barrier()

            def body(j, carry):
                pltpu.async_copy(h_hbm.at[idx_v.at[j, 0]], rows_v, gsem).wait()
                pltpu.sync_copy(rows_v, acc.at[idx_v.at[j, 1]], add=True)
                return carry

            lax.fori_loop(0, n_ch, body, 0)

        @pl.when(c == SLOW_C)
        def _():
            run(0, CH_A)

        @pl.when(c != SLOW_C)
        def _():
            run(CH_A, CH_B)
        plsc.subcore_barrier()
        pltpu.sync_copy(acc.at[pl.ds(s * RPT, RPT)],
                        out_hbm.at[c, pl.ds(s * RPT, RPT)])

    return k(h, ei4, zrows)


def _tc_finish(p0, p1, wt, b2, g2, be2):
    def body(p0_ref, p1_ref, wt_ref, b_ref, g_ref, be_ref, o_ref):
        agg = p0_ref[...] + p1_ref[...]
        h2 = jnp.dot(agg, wt_ref[...], preferred_element_type=jnp.float32)
        h2 = h2 + b_ref[...]
        mean = jnp.mean(h2, axis=0, keepdims=True)
        ctr = h2 - mean
        var = jnp.mean(ctr * ctr, axis=0, keepdims=True)
        o_ref[...] = g_ref[...] * ctr * lax.rsqrt(var + EPS) + be_ref[...]

    return pl.pallas_call(
        body,
        out_shape=jax.ShapeDtypeStruct((N_NODES, FEATS), jnp.float32),
    )(p0, p1, wt, b2, g2, be2)


def kernel(feature, edge_index, W, b, gamma, beta):
    fx = jnp.pad(feature, ((0, NPADF - N_NODES), (0, 0)))
    h = _leaky_relu_tc(fx)
    ei = edge_index.astype(jnp.int32)
    pad = NS * (CH_A + CH_B) * CHUNK - N_EDGES
    src_p = jnp.concatenate(
        [ei[0], jnp.full((pad,), N_NODES, jnp.int32)]).reshape(
            NS, CH_A + CH_B, 1, CHUNK)
    dummy = jnp.arange(pad, dtype=jnp.int32) % N_NODES
    dst_p = jnp.concatenate([ei[1], dummy]).reshape(NS, CH_A + CH_B, 1, CHUNK)
    ei4 = jnp.concatenate([src_p, dst_p], axis=2)
    zrows = jnp.zeros((RPT, FEATS), jnp.float32)
    parts = _sc_segment_sum(h, ei4, zrows)
    p0 = parts[0, :N_NODES]
    p1 = parts[1, :N_NODES]
    return _tc_finish(p0, p1, W.T,
                      b.reshape(1, FEATS),
                      gamma.reshape(1, FEATS),
                      beta.reshape(1, FEATS))


# speedup vs baseline: 1.8688x; 1.7591x over previous
import functools

import jax
import jax.numpy as jnp
from jax import lax
from jax.experimental import pallas as pl
from jax.experimental.pallas import tpu as pltpu
from jax.experimental.pallas import tpu_sc as plsc

N_NODES = 10000
FEATS = 128
N_EDGES = 320000
EPS = 1e-5

NC = 2
NS = 16
NW = NC * NS
CHUNK = 128
CHUNKS = 79
E_PAD = NW * CHUNKS * CHUNK
ROWS = 10240
RPT = ROWS // NS
DUMMY = N_NODES


def _leaky_relu_tc(x):
    def body(x_ref, o_ref):
        v = x_ref[...]
        o_ref[...] = jnp.where(v > 0, v, jnp.float32(0.2) * v)

    return pl.pallas_call(
        body,
        out_shape=jax.ShapeDtypeStruct(x.shape, x.dtype),
    )(x)


def _sc_segment_sum(h, src3, dst3, zrows):
    mesh = plsc.VectorSubcoreMesh(core_axis_name="c", subcore_axis_name="s")

    @functools.partial(
        pl.kernel,
        mesh=mesh,
        out_type=jax.ShapeDtypeStruct((NC, ROWS, FEATS), jnp.float32),
        scratch_types=[
            pltpu.VMEM((CHUNKS, CHUNK), jnp.int32),
            pltpu.VMEM((CHUNKS, CHUNK), jnp.int32),
            pltpu.VMEM((CHUNK, FEATS), jnp.float32),
            pltpu.VMEM_SHARED((ROWS, FEATS), jnp.float32),
            pltpu.SemaphoreType.DMA,
        ],
    )
    def k(h_hbm, src_hbm, dst_hbm, z_hbm, out_hbm, src_v, dst_v, rows_v, acc, sem):
        c = lax.axis_index("c")
        s = lax.axis_index("s")
        wid = s * NC + c
        pltpu.sync_copy(z_hbm, acc.at[pl.ds(s * RPT, RPT)])
        pltpu.sync_copy(src_hbm.at[wid], src_v)
        pltpu.sync_copy(dst_hbm.at[wid], dst_v)
        plsc.subcore_barrier()

        def body(j, carry):
            pltpu.async_copy(h_hbm.at[src_v.at[j]], rows_v, sem).wait()
            pltpu.sync_copy(rows_v, acc.at[dst_v.at[j]], add=True)
            return carry

        lax.fori_loop(0, CHUNKS, body, 0)
        plsc.subcore_barrier()
        pltpu.sync_copy(acc.at[pl.ds(s * RPT, RPT)],
                        out_hbm.at[c, pl.ds(s * RPT, RPT)])

    return k(h, src3, dst3, zrows)


def _tc_finish(p0, p1, wt, b2, g2, be2):
    def body(p0_ref, p1_ref, wt_ref, b_ref, g_ref, be_ref, o_ref):
        agg = p0_ref[...] + p1_ref[...]
        h2 = jnp.dot(agg, wt_ref[...], preferred_element_type=jnp.float32)
        h2 = h2 + b_ref[...]
        mean = jnp.mean(h2, axis=0, keepdims=True)
        ctr = h2 - mean
        var = jnp.mean(ctr * ctr, axis=0, keepdims=True)
        o_ref[...] = g_ref[...] * ctr * lax.rsqrt(var + EPS) + be_ref[...]

    return pl.pallas_call(
        body,
        out_shape=jax.ShapeDtypeStruct((N_NODES, FEATS), jnp.float32),
    )(p0, p1, wt, b2, g2, be2)


def kernel(feature, edge_index, W, b, gamma, beta):
    h = _leaky_relu_tc(feature)
    ei = edge_index.astype(jnp.int32)
    pad = E_PAD - N_EDGES
    src_p = jnp.concatenate(
        [ei[0], jnp.zeros((pad,), jnp.int32)]).reshape(NW, CHUNKS, CHUNK)
    dummy = DUMMY + jnp.arange(pad, dtype=jnp.int32) % (ROWS - N_NODES)
    dst_p = jnp.concatenate([ei[1], dummy]).reshape(NW, CHUNKS, CHUNK)
    zrows = jnp.zeros((RPT, FEATS), jnp.float32)
    parts = _sc_segment_sum(h, src_p, dst_p, zrows)
    p0 = parts[0, :N_NODES]
    p1 = parts[1, :N_NODES]
    return _tc_finish(p0, p1, W.T,
                      b.reshape(1, FEATS),
                      gamma.reshape(1, FEATS),
                      beta.reshape(1, FEATS))
